# baseline (speedup 1.0000x reference)
import functools

import jax
import jax.numpy as jnp
from jax.experimental import pallas as pl

_SAMPLING_EPS = 1e-5
_ROWS_PER_BLOCK = 8
_VALUE_ITERS = 30
_HI0 = 0x3F800001
_LANES = 128


def _sampler_body(vocab, index_iters, logits_ref, t_ref, p_ref, k_ref, out_ref):
    rows, vpad = out_ref.shape
    nv = vpad // _LANES

    t = t_ref[...]
    topp = p_ref[...]
    kk = jnp.clip(k_ref[...] + 1, 1, vocab)

    t = jnp.where(t < _SAMPLING_EPS, 1.0, t)
    r = 1.0 / t
    l = logits_ref[...]
    m = jnp.max(l, axis=1, keepdims=True) * r
    e3 = jnp.exp(l * r - m).reshape(rows, nv, _LANES)
    out_ref[...] = e3.reshape(rows, vpad)

    ones = jnp.ones((_LANES, _LANES), jnp.float32)

    def _allsum(part):
        return jax.lax.dot(part, ones,
                           precision=jax.lax.Precision.HIGHEST,
                           preferred_element_type=jnp.float32)

    def _rowsum3(x3):
        return _allsum(jnp.sum(x3, axis=1))

    z = _rowsum3(e3)
    tz = jnp.broadcast_to(topp, (rows, _LANES)) * z
    kk_f = jnp.broadcast_to(kk.astype(jnp.float32), (rows, _LANES))

    def value_step(_, carry):
        lo, hi = carry
        mid = (lo + hi) >> 1
        p3 = out_ref[...].reshape(rows, nv, _LANES)
        pb3 = jax.lax.bitcast_convert_type(p3, jnp.int32)
        gt3 = pb3 > mid[:, None, :]
        c = _rowsum3(jnp.where(gt3, 1.0, 0.0))
        s = _rowsum3(jnp.where(gt3, p3, 0.0))
        more = jnp.logical_and(c < kk_f, s <= tz)
        return jnp.where(more, lo, mid), jnp.where(more, mid, hi)

    lo0 = jnp.full((rows, _LANES), -1, jnp.int32)
    hi0 = jnp.full((rows, _LANES), _HI0, jnp.int32)
    _, hi = jax.lax.fori_loop(0, _VALUE_ITERS, value_step, (lo0, hi0))

    p3 = out_ref[...].reshape(rows, nv, _LANES)
    pb3 = jax.lax.bitcast_convert_type(p3, jnp.int32)
    hi3 = hi[:, None, :]
    gt3 = pb3 > hi3
    c_hi_f = _rowsum3(jnp.where(gt3, 1.0, 0.0))
    c_hi = c_hi_f.astype(jnp.int32)
    s_hi = _rowsum3(jnp.where(gt3, p3, 0.0))
    eq3 = pb3 == hi3
    ties = _rowsum3(jnp.where(eq3, 1.0, 0.0)).astype(jnp.int32)
    c_lo = c_hi + ties

    v = jax.lax.bitcast_convert_type(hi, jnp.float32)
    delta_f = jnp.where(v > 0.0, (tz - s_hi) / v, 1.0e9)
    delta = jnp.minimum(jnp.floor(delta_f), 1.0e9).astype(jnp.int32) + 1
    kk_rep = jnp.broadcast_to(kk, (rows, _LANES))
    n = jnp.clip(jnp.minimum(kk_rep, c_hi + delta), c_hi + 1, c_lo)
    m_ties = n - c_hi
    m_ties_f = m_ties.astype(jnp.float32)

    jj = jax.lax.broadcasted_iota(jnp.int32, (rows, nv, _LANES), 1)
    ll = jax.lax.broadcasted_iota(jnp.int32, (rows, nv, _LANES), 2)
    idx3 = jj * _LANES + ll

    def _split_ties():
        def index_step(_, carry):
            loj, hij = carry
            midj = (loj + hij) >> 1
            pq3 = jax.lax.bitcast_convert_type(
                out_ref[...].reshape(rows, nv, _LANES), jnp.int32)
            hit3 = jnp.logical_and(pq3 == hi3, idx3 < midj[:, None, :])
            cnt = _rowsum3(jnp.where(hit3, 1.0, 0.0))
            ge = cnt >= m_ties_f
            return jnp.where(ge, loj, midj), jnp.where(ge, midj, hij)

        loj0 = jnp.zeros((rows, _LANES), jnp.int32)
        hij0 = jnp.full((rows, _LANES), vpad, jnp.int32)
        return jax.lax.fori_loop(0, index_iters, index_step, (loj0, hij0))[1]

    j0 = jax.lax.cond(jnp.all(m_ties == ties),
                      lambda: jnp.full((rows, _LANES), vpad, jnp.int32),
                      _split_ties)

    p3 = out_ref[...].reshape(rows, nv, _LANES)
    pb3 = jax.lax.bitcast_convert_type(p3, jnp.int32)
    keep3 = jnp.logical_or(pb3 > hi3,
                           jnp.logical_and(pb3 == hi3, idx3 < j0[:, None, :]))
    masked3 = jnp.where(keep3, p3, 0.0)
    denom = jnp.maximum(_rowsum3(masked3), 1e-20)
    out_ref[...] = (masked3 / denom[:, None, :]).reshape(rows, vpad)


@jax.jit
def kernel(logits, temperatures, top_ps, top_ks):
    b, v = logits.shape
    vpad = pl.cdiv(v, _LANES) * _LANES
    if vpad != v:
        logits = jnp.pad(logits, ((0, 0), (0, vpad - v)),
                         constant_values=-jnp.inf)
    r = _ROWS_PER_BLOCK
    index_iters = max(1, vpad.bit_length())
    body = functools.partial(_sampler_body, v, index_iters)
    out = pl.pallas_call(
        body,
        grid=(b // r,),
        in_specs=[
            pl.BlockSpec((r, vpad), lambda i: (i, 0)),
            pl.BlockSpec((r, 1), lambda i: (i, 0)),
            pl.BlockSpec((r, 1), lambda i: (i, 0)),
            pl.BlockSpec((r, 1), lambda i: (i, 0)),
        ],
        out_specs=pl.BlockSpec((r, vpad), lambda i: (i, 0)),
        out_shape=jax.ShapeDtypeStruct((b, vpad), jnp.float32),
    )(logits,
      temperatures.reshape(b, 1).astype(jnp.float32),
      top_ps.reshape(b, 1).astype(jnp.float32),
      top_ks.reshape(b, 1).astype(jnp.int32))
    return out[:, :v]


# speedup vs baseline: 1.6894x; 1.6894x over previous
import functools

import jax
import jax.numpy as jnp
from jax.experimental import pallas as pl

_SAMPLING_EPS = 1e-5
_ROWS_PER_BLOCK = 16
_VALUE_ITERS = 30
_HI0 = 0x3F800001


def _sampler_body(vocab, index_iters, logits_ref, t_ref, p_ref, k_ref, out_ref):
    rows, vpad = out_ref.shape

    t = t_ref[...]
    topp = p_ref[...]
    kk = jnp.clip(k_ref[...] + 1, 1, vocab)

    t = jnp.where(t < _SAMPLING_EPS, 1.0, t)
    r = 1.0 / t
    l = logits_ref[...]
    m = jnp.max(l, axis=1, keepdims=True) * r
    e = jnp.exp(l * r - m)
    out_ref[...] = e

    def _rowsum(x):
        return jnp.sum(x, axis=1, keepdims=True)

    z = _rowsum(e)
    tz = topp * z
    kk_f = kk.astype(jnp.float32)

    def value_step(_, carry):
        lo, hi = carry
        mid = (lo + hi) >> 1
        p = out_ref[...]
        pb = jax.lax.bitcast_convert_type(p, jnp.int32)
        gt = pb > mid
        c = _rowsum(jnp.where(gt, 1.0, 0.0))
        s = _rowsum(jnp.where(gt, p, 0.0))
        more = jnp.logical_and(c < kk_f, s <= tz)
        return jnp.where(more, lo, mid), jnp.where(more, mid, hi)

    lo0 = jnp.full((rows, 1), -1, jnp.int32)
    hi0 = jnp.full((rows, 1), _HI0, jnp.int32)
    _, hi = jax.lax.fori_loop(0, _VALUE_ITERS, value_step, (lo0, hi0))

    p = out_ref[...]
    pb = jax.lax.bitcast_convert_type(p, jnp.int32)
    gt = pb > hi
    c_hi = _rowsum(jnp.where(gt, 1.0, 0.0)).astype(jnp.int32)
    s_hi = _rowsum(jnp.where(gt, p, 0.0))
    eq = pb == hi
    ties = _rowsum(jnp.where(eq, 1.0, 0.0)).astype(jnp.int32)
    c_lo = c_hi + ties

    v = jax.lax.bitcast_convert_type(hi, jnp.float32)
    delta_f = jnp.where(v > 0.0, (tz - s_hi) / v, 1.0e9)
    delta = jnp.minimum(jnp.floor(delta_f), 1.0e9).astype(jnp.int32) + 1
    n = jnp.clip(jnp.minimum(kk, c_hi + delta), c_hi + 1, c_lo)
    m_ties = n - c_hi
    m_ties_f = m_ties.astype(jnp.float32)

    iota = jax.lax.broadcasted_iota(jnp.int32, (rows, vpad), 1)

    def _split_ties():
        def index_step(_, carry):
            loj, hij = carry
            midj = (loj + hij) >> 1
            pbb = jax.lax.bitcast_convert_type(out_ref[...], jnp.int32)
            hit = jnp.logical_and(pbb == hi, iota < midj)
            cnt = _rowsum(jnp.where(hit, 1.0, 0.0))
            ge = cnt >= m_ties_f
            return jnp.where(ge, loj, midj), jnp.where(ge, midj, hij)

        loj0 = jnp.zeros((rows, 1), jnp.int32)
        hij0 = jnp.full((rows, 1), vpad, jnp.int32)
        return jax.lax.fori_loop(0, index_iters, index_step, (loj0, hij0))[1]

    j0 = jax.lax.cond(jnp.all(m_ties == ties),
                      lambda: jnp.full((rows, 1), vpad, jnp.int32),
                      _split_ties)

    p = out_ref[...]
    pb = jax.lax.bitcast_convert_type(p, jnp.int32)
    keep = jnp.logical_or(pb > hi, jnp.logical_and(pb == hi, iota < j0))
    masked = jnp.where(keep, p, 0.0)
    denom = _rowsum(masked)
    out_ref[...] = masked / jnp.maximum(denom, 1e-20)


@jax.jit
def kernel(logits, temperatures, top_ps, top_ks):
    b, v = logits.shape
    vpad = pl.cdiv(v, 128) * 128
    if vpad != v:
        logits = jnp.pad(logits, ((0, 0), (0, vpad - v)),
                         constant_values=-jnp.inf)
    r = _ROWS_PER_BLOCK
    index_iters = max(1, vpad.bit_length())
    body = functools.partial(_sampler_body, v, index_iters)
    out = pl.pallas_call(
        body,
        grid=(b // r,),
        in_specs=[
            pl.BlockSpec((r, vpad), lambda i: (i, 0)),
            pl.BlockSpec((r, 1), lambda i: (i, 0)),
            pl.BlockSpec((r, 1), lambda i: (i, 0)),
            pl.BlockSpec((r, 1), lambda i: (i, 0)),
        ],
        out_specs=pl.BlockSpec((r, vpad), lambda i: (i, 0)),
        out_shape=jax.ShapeDtypeStruct((b, vpad), jnp.float32),
    )(logits,
      temperatures.reshape(b, 1).astype(jnp.float32),
      top_ps.reshape(b, 1).astype(jnp.float32),
      top_ks.reshape(b, 1).astype(jnp.int32))
    return out[:, :v]
